# Initial kernel scaffold; baseline (speedup 1.0000x reference)
import jax
import jax.numpy as jnp
from jax.experimental import pallas as pl


def kernel(x, lap_down_indices, lap_down_values, lap_up_indices, lap_up_values, signal_belongings, W_in, b_in, W_layers, W_ro, b_ro):
    raise NotImplementedError("write your pallas kernel here")


# speedup vs baseline: 1.8191x; 1.8191x over previous
import functools

import jax
import jax.numpy as jnp
from jax import lax
from jax.experimental import pallas as pl
from jax.experimental.pallas import tpu as pltpu
from jax.experimental.pallas import tpu_sc as plsc

N_PAD = 50048
C_H = 64
C_HALF = 32
N_GRAPHS = 128
NUM_CORES = 2
NUM_SUBCORES = 16
CHUNK = 128
SEG = 4
SPAN = N_PAD // NUM_SUBCORES
HALF_ROWS = N_PAD // 2
ACC_ROWS = HALF_ROWS + 8
ZROWS = 392
TC_BLK = 3128


def _pad_edges(idx, val):
    e = idx.shape[1]
    chks_total = -(-e // (NUM_SUBCORES * CHUNK))
    chks_total = -(-chks_total // SEG) * SEG
    e_pad = NUM_SUBCORES * chks_total * CHUNK
    pad = e_pad - e
    rows = jnp.concatenate([idx[0], jnp.zeros((pad,), jnp.int32)])
    cols = jnp.concatenate([idx[1], jnp.zeros((pad,), jnp.int32)])
    vals = jnp.concatenate([val, jnp.zeros((pad,), jnp.float32)])
    shape = (NUM_SUBCORES, SEG, chks_total // SEG, CHUNK)
    vshape = (NUM_SUBCORES, SEG, (chks_total // SEG) * CHUNK)
    return rows.reshape(shape), cols.reshape(shape), vals.reshape(vshape)


def _spmm_body(rows4, cols4, vals4, tab_lo, tab_hi, out_lo, out_hi,
               rows_v, rloc_v, cols_v, vals_v, gbuf, zbuf, acc, sem, *, chk):
    c = lax.axis_index("c")
    s = lax.axis_index("s")

    def zb(i, carry):
        zbuf[i, pl.ds(0, 16)] = jnp.zeros((16,), jnp.float32)
        zbuf[i, pl.ds(16, 16)] = jnp.zeros((16,), jnp.float32)
        return carry

    lax.fori_loop(0, ZROWS, zb, 0)

    def zero_acc():
        zbase = s * 1568
        nz = jnp.where(s < 15, 4, 3)

        def zc(k, carry):
            pltpu.sync_copy(zbuf, acc.at[pl.ds(zbase + k * ZROWS, ZROWS)])
            return carry

        lax.fori_loop(0, nz, zc, 0)

        @pl.when(s == 15)
        def _():
            pltpu.sync_copy(zbuf.at[pl.ds(0, 336)],
                            acc.at[pl.ds(zbase + 3 * ZROWS, 336)])

    def run(tab, out):
        for r in range(2):
            off = r * HALF_ROWS
            zero_acc()
            plsc.subcore_barrier()

            def seg_loop(seg, carry):
                pltpu.sync_copy(rows4.at[s, seg], rows_v)
                pltpu.sync_copy(cols4.at[s, seg], cols_v)
                pltpu.sync_copy(vals4.at[s, seg], vals_v)

                def chunk_loop(j, carry2):
                    gather = pltpu.async_copy(
                        tab.at[cols_v.at[j]], gbuf, sem)

                    def rloc(q, carry3):
                        rv = rows_v[j, pl.ds(q * 16, 16)]
                        lv = rv - off
                        bad = (lv < 0) | (lv >= HALF_ROWS)
                        lv = jnp.where(bad, HALF_ROWS, lv)
                        rloc_v[j, pl.ds(q * 16, 16)] = lv
                        return carry3

                    lax.fori_loop(0, CHUNK // 16, rloc, 0)
                    gather.wait()

                    def escale(e2, carry3):
                        vsp = plsc.load_gather(
                            vals_v,
                            [jnp.full((16,), j * CHUNK + e2, jnp.int32)])
                        gbuf[e2, pl.ds(0, 16)] = gbuf[e2, pl.ds(0, 16)] * vsp
                        gbuf[e2, pl.ds(16, 16)] = (
                            gbuf[e2, pl.ds(16, 16)] * vsp)
                        return carry3

                    lax.fori_loop(0, CHUNK, escale, 0)
                    pltpu.sync_copy(gbuf, acc.at[rloc_v.at[j]], add=True)
                    return carry2

                lax.fori_loop(0, chk, chunk_loop, 0)
                return carry

            lax.fori_loop(0, SEG, seg_loop, 0)
            plsc.subcore_barrier()

            @pl.when(s // 8 == r)
            def _():
                gbase = s * SPAN
                pltpu.sync_copy(acc.at[pl.ds(gbase - off, SPAN)],
                                out.at[pl.ds(gbase, SPAN)])

            plsc.subcore_barrier()

    @pl.when(c == 0)
    def _():
        run(tab_lo, out_lo)

    @pl.when(c == 1)
    def _():
        run(tab_hi, out_hi)


def _spmm(rows4, cols4, vals4, tab_lo, tab_hi):
    chk = rows4.shape[2]
    mesh = plsc.VectorSubcoreMesh(
        core_axis_name="c", subcore_axis_name="s",
        num_cores=NUM_CORES, num_subcores=NUM_SUBCORES)
    f = pl.kernel(
        functools.partial(_spmm_body, chk=chk),
        out_type=(jax.ShapeDtypeStruct((N_PAD, C_HALF), jnp.float32),
                  jax.ShapeDtypeStruct((N_PAD, C_HALF), jnp.float32)),
        mesh=mesh,
        scratch_types=[
            pltpu.VMEM((chk, CHUNK), jnp.int32),
            pltpu.VMEM((chk, CHUNK), jnp.int32),
            pltpu.VMEM((chk, CHUNK), jnp.int32),
            pltpu.VMEM((chk * CHUNK,), jnp.float32),
            pltpu.VMEM((CHUNK, C_HALF), jnp.float32),
            pltpu.VMEM((ZROWS, C_HALF), jnp.float32),
            pltpu.VMEM_SHARED((ACC_ROWS, C_HALF), jnp.float32),
            pltpu.SemaphoreType.DMA,
        ],
        compiler_params=pltpu.CompilerParams(
            needs_layout_passes=False, use_tc_tiling_on_sc=False),
    )
    return f(rows4, cols4, vals4, tab_lo, tab_hi)


def _proj_body(x_ref, w_ref, b_ref, lo_ref, hi_ref):
    h = jnp.dot(x_ref[...], w_ref[...],
                preferred_element_type=jnp.float32) + b_ref[0:1, :]
    lo_ref[...] = h[:, :C_HALF]
    hi_ref[...] = h[:, C_HALF:]


def _input_proj(x, w_in, b_in):
    nblk = N_PAD // TC_BLK
    return pl.pallas_call(
        _proj_body,
        grid=(nblk,),
        in_specs=[
            pl.BlockSpec((TC_BLK, C_H), lambda i: (i, 0)),
            pl.BlockSpec((C_H, C_H), lambda i: (0, 0)),
            pl.BlockSpec((8, C_H), lambda i: (0, 0)),
        ],
        out_specs=[
            pl.BlockSpec((TC_BLK, C_HALF), lambda i: (i, 0)),
            pl.BlockSpec((TC_BLK, C_HALF), lambda i: (i, 0)),
        ],
        out_shape=[jax.ShapeDtypeStruct((N_PAD, C_HALF), jnp.float32)] * 2,
    )(x, w_in, jnp.broadcast_to(b_in[None, :], (8, C_H)))


def _combine_body(*refs):
    feats = refs[:10]
    w_ref = refs[10]
    lo_ref, hi_ref = refs[11], refs[12]
    cat = jnp.concatenate([f[...] for f in feats], axis=1)
    h = jnp.dot(cat, w_ref[...], preferred_element_type=jnp.float32)
    lo_ref[...] = h[:, :C_HALF]
    hi_ref[...] = h[:, C_HALF:]


def _combine(feats, w_flat):
    nblk = N_PAD // TC_BLK
    return pl.pallas_call(
        _combine_body,
        grid=(nblk,),
        in_specs=[pl.BlockSpec((TC_BLK, C_HALF), lambda i: (i, 0))] * 10
        + [pl.BlockSpec((5 * C_H, C_H), lambda i: (0, 0))],
        out_specs=[
            pl.BlockSpec((TC_BLK, C_HALF), lambda i: (i, 0)),
            pl.BlockSpec((TC_BLK, C_HALF), lambda i: (i, 0)),
        ],
        out_shape=[jax.ShapeDtypeStruct((N_PAD, C_HALF), jnp.float32)] * 2,
    )(*feats, w_flat)


def _readout_body(lo_ref, hi_ref, wro_ref, bro_ref, sig_ref, out_ref, acc):
    i = pl.program_id(0)

    @pl.when(i == 0)
    def _():
        acc[...] = jnp.zeros_like(acc)

    y = (jnp.dot(lo_ref[...], wro_ref[:C_HALF, :],
                 preferred_element_type=jnp.float32)
         + jnp.dot(hi_ref[...], wro_ref[C_HALF:, :],
                   preferred_element_type=jnp.float32)
         + bro_ref[0:1, :])
    ids = sig_ref[0, 0, :]
    iota = lax.broadcasted_iota(jnp.int32, (TC_BLK, N_GRAPHS), 1)
    oh = jnp.where(ids[:, None] == iota, 1.0, 0.0).astype(jnp.float32)
    cat = jnp.concatenate(
        [y, jnp.ones((TC_BLK, C_HALF), jnp.float32)], axis=1)
    acc[...] += lax.dot_general(
        oh, cat, (((0,), (0,)), ((), ())),
        preferred_element_type=jnp.float32)

    @pl.when(i == pl.num_programs(0) - 1)
    def _():
        sums = acc[:, :C_HALF]
        cnts = acc[:, C_HALF:]
        out_ref[...] = jnp.where(cnts > 0.0, sums / cnts, 0.0)


def _readout(h_lo, h_hi, w_ro, b_ro, sig_pad):
    nblk = N_PAD // TC_BLK
    c_out = w_ro.shape[1]
    sig3 = sig_pad.reshape(nblk, 1, TC_BLK)
    return pl.pallas_call(
        _readout_body,
        grid=(nblk,),
        in_specs=[
            pl.BlockSpec((TC_BLK, C_HALF), lambda i: (i, 0)),
            pl.BlockSpec((TC_BLK, C_HALF), lambda i: (i, 0)),
            pl.BlockSpec((C_H, c_out), lambda i: (0, 0)),
            pl.BlockSpec((8, c_out), lambda i: (0, 0)),
            pl.BlockSpec((1, 1, TC_BLK), lambda i: (i, 0, 0)),
        ],
        out_specs=pl.BlockSpec((N_GRAPHS, c_out), lambda i: (0, 0)),
        out_shape=jax.ShapeDtypeStruct((N_GRAPHS, c_out), jnp.float32),
        scratch_shapes=[pltpu.VMEM((N_GRAPHS, 2 * c_out), jnp.float32)],
    )(h_lo, h_hi, w_ro, jnp.broadcast_to(b_ro[None, :], (8, c_out)), sig3)


def kernel(x, lap_down_indices, lap_down_values, lap_up_indices,
           lap_up_values, signal_belongings, W_in, b_in, W_layers,
           W_ro, b_ro):
    n = x.shape[0]
    rd, cd, vd = _pad_edges(lap_down_indices, lap_down_values)
    ru, cu, vu = _pad_edges(lap_up_indices, lap_up_values)
    x_pad = jnp.pad(x, ((0, N_PAD - n), (0, 0)))
    sig_pad = jnp.pad(signal_belongings, (0, N_PAD - n),
                      constant_values=N_GRAPHS)
    w_flat = jnp.transpose(W_layers, (0, 3, 1, 2)).reshape(
        W_layers.shape[0], 5 * C_H, C_H)
    h_lo, h_hi = _input_proj(x_pad, W_in, b_in)
    for l in range(W_layers.shape[0]):
        d1_lo, d1_hi = _spmm(rd, cd, vd, h_lo, h_hi)
        d2_lo, d2_hi = _spmm(rd, cd, vd, d1_lo, d1_hi)
        u1_lo, u1_hi = _spmm(ru, cu, vu, h_lo, h_hi)
        u2_lo, u2_hi = _spmm(ru, cu, vu, u1_lo, u1_hi)
        h_lo, h_hi = _combine(
            (h_lo, h_hi, d1_lo, d1_hi, d2_lo, d2_hi,
             u1_lo, u1_hi, u2_lo, u2_hi), w_flat[l])
    return _readout(h_lo, h_hi, W_ro, b_ro, sig_pad)


# speedup vs baseline: 3.4495x; 1.8963x over previous
import functools

import jax
import jax.numpy as jnp
from jax import lax
from jax.experimental import pallas as pl
from jax.experimental.pallas import tpu as pltpu
from jax.experimental.pallas import tpu_sc as plsc

N_PAD = 50048
C_H = 64
C_HALF = 32
N_GRAPHS = 128
NUM_CORES = 2
NUM_SUBCORES = 16
CHUNK = 128
SEG = 4
SPAN = N_PAD // NUM_SUBCORES
ZROWS = 392
TC_BLK = 3128


def _pad_edges(idx, val):
    e = idx.shape[1]
    chks_total = -(-e // (NUM_SUBCORES * CHUNK))
    chks_total = -(-chks_total // SEG) * SEG
    e_pad = NUM_SUBCORES * chks_total * CHUNK
    pad = e_pad - e
    rows = jnp.concatenate([idx[0], jnp.zeros((pad,), jnp.int32)])
    cols = jnp.concatenate([idx[1], jnp.zeros((pad,), jnp.int32)])
    vals = jnp.concatenate([val, jnp.zeros((pad,), jnp.float32)])
    shape = (NUM_SUBCORES, SEG, chks_total // SEG, CHUNK)
    vshape = (NUM_SUBCORES, SEG, (chks_total // SEG) * CHUNK)
    return rows.reshape(shape), cols.reshape(vshape), vals.reshape(vshape)


NBUF = 7
C_Q = 16


def _spmm_body(rows4, cols3, vals3, tab_lo, tab_hi, out_lo, out_hi,
               rows_v, cols_v, vals_v, gbuf, sbuf, zbuf, acc,
               *sems, chk):
    c = lax.axis_index("c")
    s = lax.axis_index("s")
    semg = sems[:NBUF]
    sems = sems[NBUF:]
    ngrp = chk // NBUF

    def zb(i, carry):
        zbuf[i, pl.ds(0, 16)] = jnp.zeros((16,), jnp.float32)
        return carry

    lax.fori_loop(0, ZROWS, zb, 0)
    base = s * SPAN

    def zero_acc():
        for k in range(SPAN // ZROWS):
            pltpu.sync_copy(zbuf, acc.at[pl.ds(base + k * ZROWS, ZROWS)])
        rem = SPAN - (SPAN // ZROWS) * ZROWS
        if rem:
            pltpu.sync_copy(
                zbuf.at[pl.ds(0, rem)],
                acc.at[pl.ds(base + (SPAN // ZROWS) * ZROWS, rem)])

    def fire_gather(tab, j, b):
        return pltpu.async_copy(
            tab.at[cols_v.at[pl.ds(j * CHUNK, CHUNK)]],
            gbuf.at[b], semg[b])

    def fire_scatter(j, b):
        return pltpu.async_copy(
            sbuf.at[b], acc.at[rows_v.at[j]], sems[b], add=True)

    def run(tab, out):
        for r in range(2):
            zero_acc()
            plsc.subcore_barrier()

            def seg_loop(seg, carry):
                pltpu.sync_copy(rows4.at[s, seg], rows_v)
                pltpu.sync_copy(cols3.at[s, seg], cols_v)
                pltpu.sync_copy(vals3.at[s, seg], vals_v)

                def ctrans(i, carry2):
                    for u in range(4):
                        ii = i * 4 + u
                        cv = cols_v[pl.ds(ii * 16, 16)]
                        cols_v[pl.ds(ii * 16, 16)] = cv * 2 + r
                    return carry2

                lax.fori_loop(0, chk * 8 // 4, ctrans, 0)

                for b in range(NBUF):
                    fire_gather(tab, b, b)

                def group(g, carry2):
                    for b in range(NBUF):
                        j = g * NBUF + b
                        pltpu.make_async_copy(
                            tab.at[cols_v.at[pl.ds(j * CHUNK, CHUNK)]],
                            gbuf.at[b], semg[b]).wait()

                        @pl.when(g > 0)
                        def _():
                            pltpu.make_async_copy(
                                sbuf.at[b],
                                acc.at[rows_v.at[j - NBUF]],
                                sems[b]).wait()

                        jbase = j * CHUNK

                        def escale(e0, carry3):
                            for u in range(8):
                                e2 = e0 * 8 + u
                                vsp = plsc.load_gather(
                                    vals_v,
                                    [jnp.full((16,), jbase + e2, jnp.int32)])
                                sbuf[b, e2, pl.ds(0, 16)] = (
                                    gbuf[b, e2, pl.ds(0, 16)] * vsp)
                            return carry3

                        lax.fori_loop(0, CHUNK // 8, escale, 0)
                        fire_scatter(j, b)

                        @pl.when(g + 1 < ngrp)
                        def _():
                            fire_gather(tab, j + NBUF, b)

                    return carry2

                lax.fori_loop(0, ngrp, group, 0)
                for b in range(NBUF):
                    pltpu.make_async_copy(
                        sbuf.at[b],
                        acc.at[rows_v.at[(ngrp - 1) * NBUF + b]],
                        sems[b]).wait()
                return carry

            lax.fori_loop(0, SEG, seg_loop, 0)
            plsc.subcore_barrier()
            pltpu.sync_copy(acc.at[pl.ds(base, SPAN)],
                            out.at[pl.ds(base, SPAN), r])
            plsc.subcore_barrier()

    @pl.when(c == 0)
    def _():
        run(tab_lo, out_lo)

    @pl.when(c == 1)
    def _():
        run(tab_hi, out_hi)


def _spmm(rows4, cols3, vals3, tab_lo, tab_hi):
    chk = rows4.shape[2]
    mesh = plsc.VectorSubcoreMesh(
        core_axis_name="c", subcore_axis_name="s",
        num_cores=NUM_CORES, num_subcores=NUM_SUBCORES)
    f = pl.kernel(
        functools.partial(_spmm_body, chk=chk),
        out_type=(jax.ShapeDtypeStruct((N_PAD, 2, C_Q), jnp.float32),
                  jax.ShapeDtypeStruct((N_PAD, 2, C_Q), jnp.float32)),
        mesh=mesh,
        scratch_types=[
            pltpu.VMEM((chk, CHUNK), jnp.int32),
            pltpu.VMEM((chk * CHUNK,), jnp.int32),
            pltpu.VMEM((chk * CHUNK,), jnp.float32),
            pltpu.VMEM((NBUF, CHUNK, C_Q), jnp.float32),
            pltpu.VMEM((NBUF, CHUNK, C_Q), jnp.float32),
            pltpu.VMEM((ZROWS, C_Q), jnp.float32),
            pltpu.VMEM_SHARED((N_PAD, C_Q), jnp.float32),
        ] + [pltpu.SemaphoreType.DMA] * (2 * NBUF),
        compiler_params=pltpu.CompilerParams(
            needs_layout_passes=False, use_tc_tiling_on_sc=False),
    )
    t_lo = tab_lo.reshape(2 * N_PAD, C_Q)
    t_hi = tab_hi.reshape(2 * N_PAD, C_Q)
    o_lo, o_hi = f(rows4, cols3, vals3, t_lo, t_hi)
    return o_lo.reshape(N_PAD, C_HALF), o_hi.reshape(N_PAD, C_HALF)


def _proj_body(x_ref, w_ref, b_ref, lo_ref, hi_ref):
    h = jnp.dot(x_ref[...], w_ref[...],
                preferred_element_type=jnp.float32) + b_ref[0:1, :]
    lo_ref[...] = h[:, :C_HALF]
    hi_ref[...] = h[:, C_HALF:]


def _input_proj(x, w_in, b_in):
    nblk = N_PAD // TC_BLK
    return pl.pallas_call(
        _proj_body,
        grid=(nblk,),
        in_specs=[
            pl.BlockSpec((TC_BLK, C_H), lambda i: (i, 0)),
            pl.BlockSpec((C_H, C_H), lambda i: (0, 0)),
            pl.BlockSpec((8, C_H), lambda i: (0, 0)),
        ],
        out_specs=[
            pl.BlockSpec((TC_BLK, C_HALF), lambda i: (i, 0)),
            pl.BlockSpec((TC_BLK, C_HALF), lambda i: (i, 0)),
        ],
        out_shape=[jax.ShapeDtypeStruct((N_PAD, C_HALF), jnp.float32)] * 2,
    )(x, w_in, jnp.broadcast_to(b_in[None, :], (8, C_H)))


def _combine_body(*refs):
    feats = refs[:10]
    w_ref = refs[10]
    lo_ref, hi_ref = refs[11], refs[12]
    cat = jnp.concatenate([f[...] for f in feats], axis=1)
    h = jnp.dot(cat, w_ref[...], preferred_element_type=jnp.float32)
    lo_ref[...] = h[:, :C_HALF]
    hi_ref[...] = h[:, C_HALF:]


def _combine(feats, w_flat):
    nblk = N_PAD // TC_BLK
    return pl.pallas_call(
        _combine_body,
        grid=(nblk,),
        in_specs=[pl.BlockSpec((TC_BLK, C_HALF), lambda i: (i, 0))] * 10
        + [pl.BlockSpec((5 * C_H, C_H), lambda i: (0, 0))],
        out_specs=[
            pl.BlockSpec((TC_BLK, C_HALF), lambda i: (i, 0)),
            pl.BlockSpec((TC_BLK, C_HALF), lambda i: (i, 0)),
        ],
        out_shape=[jax.ShapeDtypeStruct((N_PAD, C_HALF), jnp.float32)] * 2,
    )(*feats, w_flat)


def _readout_body(lo_ref, hi_ref, wro_ref, bro_ref, sig_ref, out_ref, acc):
    i = pl.program_id(0)

    @pl.when(i == 0)
    def _():
        acc[...] = jnp.zeros_like(acc)

    y = (jnp.dot(lo_ref[...], wro_ref[:C_HALF, :],
                 preferred_element_type=jnp.float32)
         + jnp.dot(hi_ref[...], wro_ref[C_HALF:, :],
                   preferred_element_type=jnp.float32)
         + bro_ref[0:1, :])
    ids = sig_ref[0, 0, :]
    iota = lax.broadcasted_iota(jnp.int32, (TC_BLK, N_GRAPHS), 1)
    oh = jnp.where(ids[:, None] == iota, 1.0, 0.0).astype(jnp.float32)
    cat = jnp.concatenate(
        [y, jnp.ones((TC_BLK, C_HALF), jnp.float32)], axis=1)
    acc[...] += lax.dot_general(
        oh, cat, (((0,), (0,)), ((), ())),
        preferred_element_type=jnp.float32)

    @pl.when(i == pl.num_programs(0) - 1)
    def _():
        sums = acc[:, :C_HALF]
        cnts = acc[:, C_HALF:]
        out_ref[...] = jnp.where(cnts > 0.0, sums / cnts, 0.0)


def _readout(h_lo, h_hi, w_ro, b_ro, sig_pad):
    nblk = N_PAD // TC_BLK
    c_out = w_ro.shape[1]
    sig3 = sig_pad.reshape(nblk, 1, TC_BLK)
    return pl.pallas_call(
        _readout_body,
        grid=(nblk,),
        in_specs=[
            pl.BlockSpec((TC_BLK, C_HALF), lambda i: (i, 0)),
            pl.BlockSpec((TC_BLK, C_HALF), lambda i: (i, 0)),
            pl.BlockSpec((C_H, c_out), lambda i: (0, 0)),
            pl.BlockSpec((8, c_out), lambda i: (0, 0)),
            pl.BlockSpec((1, 1, TC_BLK), lambda i: (i, 0, 0)),
        ],
        out_specs=pl.BlockSpec((N_GRAPHS, c_out), lambda i: (0, 0)),
        out_shape=jax.ShapeDtypeStruct((N_GRAPHS, c_out), jnp.float32),
        scratch_shapes=[pltpu.VMEM((N_GRAPHS, 2 * c_out), jnp.float32)],
    )(h_lo, h_hi, w_ro, jnp.broadcast_to(b_ro[None, :], (8, c_out)), sig3)


def kernel(x, lap_down_indices, lap_down_values, lap_up_indices,
           lap_up_values, signal_belongings, W_in, b_in, W_layers,
           W_ro, b_ro):
    n = x.shape[0]
    rd, cd, vd = _pad_edges(lap_down_indices, lap_down_values)
    ru, cu, vu = _pad_edges(lap_up_indices, lap_up_values)
    x_pad = jnp.pad(x, ((0, N_PAD - n), (0, 0)))
    sig_pad = jnp.pad(signal_belongings, (0, N_PAD - n),
                      constant_values=N_GRAPHS)
    w_flat = jnp.transpose(W_layers, (0, 3, 1, 2)).reshape(
        W_layers.shape[0], 5 * C_H, C_H)
    h_lo, h_hi = _input_proj(x_pad, W_in, b_in)
    for l in range(W_layers.shape[0]):
        d1_lo, d1_hi = _spmm(rd, cd, vd, h_lo, h_hi)
        d2_lo, d2_hi = _spmm(rd, cd, vd, d1_lo, d1_hi)
        u1_lo, u1_hi = _spmm(ru, cu, vu, h_lo, h_hi)
        u2_lo, u2_hi = _spmm(ru, cu, vu, u1_lo, u1_hi)
        h_lo, h_hi = _combine(
            (h_lo, h_hi, d1_lo, d1_hi, d2_lo, d2_hi,
             u1_lo, u1_hi, u2_lo, u2_hi), w_flat[l])
    return _readout(h_lo, h_hi, W_ro, b_ro, sig_pad)


# speedup vs baseline: 7.4592x; 2.1624x over previous
import functools

import jax
import jax.numpy as jnp
from jax import lax
from jax.experimental import pallas as pl
from jax.experimental.pallas import tpu as pltpu
from jax.experimental.pallas import tpu_sc as plsc

N_PAD = 50048
C_H = 64
C_HALF = 32
C_Q = 16
N_GRAPHS = 128
NUM_CORES = 2
NUM_SUBCORES = 16
CHUNK = 128
SEG = 4
NBUF = 7
SPAN = N_PAD // NUM_SUBCORES
ZROWS = 392
TC_BLK = 1088


def _pad_edges(idx, val):
    e = idx.shape[1]
    chks_total = -(-e // (NUM_SUBCORES * CHUNK))
    chks_total = -(-chks_total // SEG) * SEG
    e_pad = NUM_SUBCORES * chks_total * CHUNK
    pad = e_pad - e
    rows = jnp.concatenate([idx[0], jnp.zeros((pad,), jnp.int32)])
    cols = jnp.concatenate([idx[1], jnp.zeros((pad,), jnp.int32)])
    vals = jnp.concatenate([val, jnp.zeros((pad,), jnp.float32)])
    shape = (NUM_SUBCORES, SEG, chks_total // SEG, CHUNK)
    vshape = (NUM_SUBCORES, SEG, (chks_total // SEG) * CHUNK)
    return rows.reshape(shape), cols.reshape(vshape), vals.reshape(vshape)


def _layer_body(rows_d, cols_d, vals_d, rows_u, cols_u, vals_u, tab,
                o1, o2, o3, o4,
                rows_v, cols_v, vals_v, gbuf, sbuf, zbuf, acc,
                *sems, chk):
    c = lax.axis_index("c")
    s = lax.axis_index("s")
    semg = sems[:NBUF]
    sems = sems[NBUF:]
    ngrp = chk // NBUF
    base = s * SPAN

    def zb(i, carry):
        zbuf[i, pl.ds(0, 16)] = jnp.zeros((16,), jnp.float32)
        return carry

    lax.fori_loop(0, ZROWS, zb, 0)

    def zero_acc():
        for k in range(SPAN // ZROWS):
            pltpu.sync_copy(zbuf, acc.at[pl.ds(base + k * ZROWS, ZROWS)])
        rem = SPAN - (SPAN // ZROWS) * ZROWS
        if rem:
            pltpu.sync_copy(
                zbuf.at[pl.ds(0, rem)],
                acc.at[pl.ds(base + (SPAN // ZROWS) * ZROWS, rem)])

    def spmm(rows4, cols3, vals3, src, dst):
        def fire_gather(j, b):
            return pltpu.async_copy(
                src.at[cols_v.at[pl.ds(j * CHUNK, CHUNK)]],
                gbuf.at[b], semg[b])

        def sweep(r, carry0):
            offs = (2 * c + r) * N_PAD
            zero_acc()
            plsc.subcore_barrier()

            def seg_loop(seg, carry):
                pltpu.sync_copy(rows4.at[s, seg], rows_v)
                pltpu.sync_copy(cols3.at[s, seg], cols_v)
                pltpu.sync_copy(vals3.at[s, seg], vals_v)

                def ctrans(i, carry2):
                    for u in range(4):
                        ii = i * 4 + u
                        cv = cols_v[pl.ds(ii * 16, 16)]
                        cols_v[pl.ds(ii * 16, 16)] = cv + offs
                    return carry2

                lax.fori_loop(0, chk * 8 // 4, ctrans, 0)

                for b in range(NBUF):
                    fire_gather(b, b)

                def group(g, carry2):
                    for b in range(NBUF):
                        j = g * NBUF + b
                        pltpu.make_async_copy(
                            src.at[cols_v.at[pl.ds(j * CHUNK, CHUNK)]],
                            gbuf.at[b], semg[b]).wait()

                        @pl.when(g > 0)
                        def _():
                            pltpu.make_async_copy(
                                sbuf.at[b],
                                acc.at[rows_v.at[j - NBUF]],
                                sems[b]).wait()

                        jbase = j * CHUNK

                        def escale(e0, carry3):
                            vals16 = vals_v[pl.ds(jbase + e0 * 16, 16)]
                            for u in range(16):
                                e2 = e0 * 16 + u
                                vsp = vals16.at[
                                    jnp.full((16,), u, jnp.int32)].get(
                                        mode="promise_in_bounds")
                                sbuf[b, e2, pl.ds(0, 16)] = (
                                    gbuf[b, e2, pl.ds(0, 16)] * vsp)
                            return carry3

                        lax.fori_loop(0, CHUNK // 16, escale, 0)
                        pltpu.async_copy(
                            sbuf.at[b], acc.at[rows_v.at[j]],
                            sems[b], add=True)

                        @pl.when(g + 1 < ngrp)
                        def _():
                            fire_gather(j + NBUF, b)

                    return carry2

                lax.fori_loop(0, ngrp, group, 0)
                for b in range(NBUF):
                    pltpu.make_async_copy(
                        sbuf.at[b],
                        acc.at[rows_v.at[(ngrp - 1) * NBUF + b]],
                        sems[b]).wait()
                return carry

            lax.fori_loop(0, SEG, seg_loop, 0)
            plsc.subcore_barrier()
            pltpu.sync_copy(acc.at[pl.ds(base, SPAN)],
                            dst.at[pl.ds(offs + base, SPAN)])
            plsc.subcore_barrier()
            return carry0

        lax.fori_loop(0, 2, sweep, 0)

    spmm(rows_d, cols_d, vals_d, tab, o1)
    spmm(rows_d, cols_d, vals_d, o1, o2)
    spmm(rows_u, cols_u, vals_u, tab, o3)
    spmm(rows_u, cols_u, vals_u, o3, o4)


def _sc_layer(edges_d, edges_u, h4):
    chk = edges_d[0].shape[2]
    mesh = plsc.VectorSubcoreMesh(
        core_axis_name="c", subcore_axis_name="s",
        num_cores=NUM_CORES, num_subcores=NUM_SUBCORES)
    f = pl.kernel(
        functools.partial(_layer_body, chk=chk),
        out_type=tuple(
            jax.ShapeDtypeStruct((4 * N_PAD, C_Q), jnp.float32)
            for _ in range(4)),
        mesh=mesh,
        scratch_types=[
            pltpu.VMEM((chk, CHUNK), jnp.int32),
            pltpu.VMEM((chk * CHUNK,), jnp.int32),
            pltpu.VMEM((chk * CHUNK,), jnp.float32),
            pltpu.VMEM((NBUF, CHUNK, C_Q), jnp.float32),
            pltpu.VMEM((NBUF, CHUNK, C_Q), jnp.float32),
            pltpu.VMEM((ZROWS, C_Q), jnp.float32),
            pltpu.VMEM_SHARED((N_PAD, C_Q), jnp.float32),
        ] + [pltpu.SemaphoreType.DMA] * (2 * NBUF),
        compiler_params=pltpu.CompilerParams(
            needs_layout_passes=False, use_tc_tiling_on_sc=False),
    )
    return f(*edges_d, *edges_u, h4)


def _proj_body(x_ref, w_ref, b_ref, out_ref):
    h = jnp.dot(x_ref[...], w_ref[...],
                preferred_element_type=jnp.float32) + b_ref[0:1, :]
    for p in range(4):
        out_ref[p] = h[:, p * C_Q:(p + 1) * C_Q]


def _input_proj(x, w_in, b_in):
    nblk = N_PAD // TC_BLK
    return pl.pallas_call(
        _proj_body,
        grid=(nblk,),
        in_specs=[
            pl.BlockSpec((TC_BLK, C_H), lambda i: (i, 0)),
            pl.BlockSpec((C_H, C_H), lambda i: (0, 0)),
            pl.BlockSpec((8, C_H), lambda i: (0, 0)),
        ],
        out_specs=pl.BlockSpec((4, TC_BLK, C_Q), lambda i: (0, i, 0)),
        out_shape=jax.ShapeDtypeStruct((4, N_PAD, C_Q), jnp.float32),
    )(x, w_in, jnp.broadcast_to(b_in[None, :], (8, C_H)))


def _combine_body(*refs):
    feats = refs[:5]
    w_ref = refs[5]
    out_ref = refs[6]
    cat = jnp.concatenate(
        [f[p] for f in feats for p in range(4)], axis=1)
    h = jnp.dot(cat, w_ref[...], preferred_element_type=jnp.float32)
    for p in range(4):
        out_ref[p] = h[:, p * C_Q:(p + 1) * C_Q]


def _combine(feats, w_flat):
    nblk = N_PAD // TC_BLK
    return pl.pallas_call(
        _combine_body,
        grid=(nblk,),
        in_specs=[pl.BlockSpec((4, TC_BLK, C_Q), lambda i: (0, i, 0))] * 5
        + [pl.BlockSpec((5 * C_H, C_H), lambda i: (0, 0))],
        out_specs=pl.BlockSpec((4, TC_BLK, C_Q), lambda i: (0, i, 0)),
        out_shape=jax.ShapeDtypeStruct((4, N_PAD, C_Q), jnp.float32),
    )(*feats, w_flat)


def _readout_body(h_ref, wro_ref, bro_ref, sig_ref, out_ref, acc):
    i = pl.program_id(0)

    @pl.when(i == 0)
    def _():
        acc[...] = jnp.zeros_like(acc)

    h = jnp.concatenate([h_ref[p] for p in range(4)], axis=1)
    y = jnp.dot(h, wro_ref[...],
                preferred_element_type=jnp.float32) + bro_ref[0:1, :]
    ids = sig_ref[0, 0, :]
    iota = lax.broadcasted_iota(jnp.int32, (TC_BLK, N_GRAPHS), 1)
    oh = jnp.where(ids[:, None] == iota, 1.0, 0.0).astype(jnp.float32)
    cat = jnp.concatenate(
        [y, jnp.ones((TC_BLK, C_HALF), jnp.float32)], axis=1)
    acc[...] += lax.dot_general(
        oh, cat, (((0,), (0,)), ((), ())),
        preferred_element_type=jnp.float32)

    @pl.when(i == pl.num_programs(0) - 1)
    def _():
        sums = acc[:, :C_HALF]
        cnts = acc[:, C_HALF:]
        out_ref[...] = jnp.where(cnts > 0.0, sums / cnts, 0.0)


def _readout(h4, w_ro, b_ro, sig_pad):
    nblk = N_PAD // TC_BLK
    c_out = w_ro.shape[1]
    sig3 = sig_pad.reshape(nblk, 1, TC_BLK)
    return pl.pallas_call(
        _readout_body,
        grid=(nblk,),
        in_specs=[
            pl.BlockSpec((4, TC_BLK, C_Q), lambda i: (0, i, 0)),
            pl.BlockSpec((C_H, c_out), lambda i: (0, 0)),
            pl.BlockSpec((8, c_out), lambda i: (0, 0)),
            pl.BlockSpec((1, 1, TC_BLK), lambda i: (i, 0, 0)),
        ],
        out_specs=pl.BlockSpec((N_GRAPHS, c_out), lambda i: (0, 0)),
        out_shape=jax.ShapeDtypeStruct((N_GRAPHS, c_out), jnp.float32),
        scratch_shapes=[pltpu.VMEM((N_GRAPHS, 2 * c_out), jnp.float32)],
    )(h4, w_ro, jnp.broadcast_to(b_ro[None, :], (8, c_out)), sig3)


def kernel(x, lap_down_indices, lap_down_values, lap_up_indices,
           lap_up_values, signal_belongings, W_in, b_in, W_layers,
           W_ro, b_ro):
    n = x.shape[0]
    edges_d = _pad_edges(lap_down_indices, lap_down_values)
    edges_u = _pad_edges(lap_up_indices, lap_up_values)
    x_pad = jnp.pad(x, ((0, N_PAD - n), (0, 0)))
    sig_pad = jnp.pad(signal_belongings, (0, N_PAD - n),
                      constant_values=N_GRAPHS)
    w_flat = jnp.transpose(W_layers, (0, 3, 1, 2)).reshape(
        W_layers.shape[0], 5 * C_H, C_H)
    h4 = _input_proj(x_pad, W_in, b_in)
    for l in range(W_layers.shape[0]):
        d1, d2, u1, u2 = (
            o.reshape(4, N_PAD, C_Q)
            for o in _sc_layer(edges_d, edges_u,
                               h4.reshape(4 * N_PAD, C_Q)))
        h4 = _combine((h4, d1, d2, u1, u2), w_flat[l])
    return _readout(h4, W_ro, b_ro, sig_pad)
